# baseline (speedup 1.0000x reference)
import jax
import jax.numpy as jnp
from jax.experimental import pallas as pl
from jax.experimental.pallas import tpu as pltpu


def _pick_block(n, target):
    best = None
    for b in range(8, min(n, target) + 1, 8):
        if n % b == 0:
            best = b
    return best if best is not None else n


def _dot(a, b):
    return jax.lax.dot_general(
        a, b, (((1,), (0,)), ((), ())),
        preferred_element_type=jnp.float32,
        precision=jax.lax.Precision.DEFAULT)


def _support_kernel(x_ref, w_ref, o_ref):
    o_ref[...] = jax.lax.dot_general(
        x_ref[...], w_ref[...], (((1,), (0,)), ((), ())),
        preferred_element_type=jnp.float32,
        precision=jax.lax.Precision.HIGHEST).astype(jnp.bfloat16)


def _layer1_kernel(adj0_ref, adj1_ref, sup_ref, b_ref, w2_ref, out_ref,
                   *, bi):
    sup = sup_ref[...]
    for s, a_ref in enumerate((adj0_ref, adj1_ref)):
        acc = _dot(a_ref[...].astype(jnp.bfloat16), sup)
        h = jnp.maximum(acc + b_ref[...], 0.0)
        out_ref[pl.ds(s * bi, bi), :] = _dot(
            h.astype(jnp.bfloat16), w2_ref[...]).astype(jnp.bfloat16)


def _layer2_kernel(adj0_ref, adj1_ref, sup_ref, b_ref, out_ref, *, bi):
    sup = sup_ref[...]
    for s, a_ref in enumerate((adj0_ref, adj1_ref)):
        acc = _dot(a_ref[...].astype(jnp.bfloat16), sup)
        out_ref[pl.ds(s * bi, bi), :] = acc + b_ref[...]


def kernel(x, adj, W1, b1, W2, b2):
    import functools
    n, d = x.shape
    bi = _pick_block(n, 200)
    assert n % (2 * bi) == 0

    b1r = b1.reshape(1, d)
    b2r = b2.reshape(1, d)
    w2_bf = W2.astype(jnp.bfloat16)

    bs = _pick_block(n, 1000)
    support1 = pl.pallas_call(
        _support_kernel,
        grid=(n // bs,),
        in_specs=[
            pl.BlockSpec((bs, d), lambda i: (i, 0)),
            pl.BlockSpec((d, d), lambda i: (0, 0)),
        ],
        out_specs=pl.BlockSpec((bs, d), lambda i: (i, 0)),
        out_shape=jax.ShapeDtypeStruct((n, d), jnp.bfloat16),
        compiler_params=pltpu.CompilerParams(
            dimension_semantics=("arbitrary",)),
    )(x, W1)

    grid = (n // (2 * bi),)
    adj_specs = [
        pl.BlockSpec((bi, n), lambda i: (2 * i, 0)),
        pl.BlockSpec((bi, n), lambda i: (2 * i + 1, 0)),
    ]

    support2 = pl.pallas_call(
        functools.partial(_layer1_kernel, bi=bi),
        grid=grid,
        in_specs=adj_specs + [
            pl.BlockSpec((n, d), lambda i: (0, 0)),
            pl.BlockSpec((1, d), lambda i: (0, 0)),
            pl.BlockSpec((d, d), lambda i: (0, 0)),
        ],
        out_specs=pl.BlockSpec((2 * bi, d), lambda i: (i, 0)),
        out_shape=jax.ShapeDtypeStruct((n, d), jnp.bfloat16),
        compiler_params=pltpu.CompilerParams(
            dimension_semantics=("arbitrary",)),
    )(adj, adj, support1, b1r, w2_bf)

    out = pl.pallas_call(
        functools.partial(_layer2_kernel, bi=bi),
        grid=grid,
        in_specs=adj_specs + [
            pl.BlockSpec((n, d), lambda i: (0, 0)),
            pl.BlockSpec((1, d), lambda i: (0, 0)),
        ],
        out_specs=pl.BlockSpec((2 * bi, d), lambda i: (i, 0)),
        out_shape=jax.ShapeDtypeStruct((n, d), jnp.float32),
        compiler_params=pltpu.CompilerParams(
            dimension_semantics=("arbitrary",)),
    )(adj, adj, support2, b2r)

    return out
